# baseline (speedup 1.0000x reference)
import functools

import jax
import jax.numpy as jnp
from jax import lax
from jax.experimental import pallas as pl
from jax.experimental.pallas import tpu as pltpu
from jax.experimental.pallas import tpu_sc as plsc

VOCAB = 1000000
EMBED_DIM = 64
BATCH = 16384
HIST_LEN = 50

NC = 2
NS = 16
NW = NC * NS

B_TOTAL = BATCH * HIST_LEN
B_PER_W = B_TOTAL // NW
CHUNK = 128
N_CHUNKS = B_PER_W // CHUNK
K = 5
BANK = K * CHUNK
T = N_CHUNKS // K


def _gather_kernel(x_hbm, table_hbm, out_hbm, idx_v, rows_v,
                   gsem0, gsem1, ssem0, ssem1):
    wid = lax.axis_index("s") * NC + lax.axis_index("c")
    base = wid * B_PER_W
    gsems = (gsem0, gsem1)
    ssems = (ssem0, ssem1)

    pltpu.sync_copy(x_hbm.at[wid], idx_v)

    def issue_bank_gathers(t, p):
        for c in range(K):
            pltpu.async_copy(table_hbm.at[idx_v.at[t * K + c]],
                             rows_v.at[p, pl.ds(c * CHUNK, CHUNK)], gsems[p])

    def drain_bank_gathers(p):
        pltpu.make_async_copy(table_hbm.at[pl.ds(0, BANK)], rows_v.at[p],
                              gsems[p]).wait()

    def issue_bank_scatter(t, p):
        pltpu.async_copy(rows_v.at[p], out_hbm.at[pl.ds(base + t * BANK, BANK)],
                         ssems[p])

    def wait_bank_scatter(p):
        pltpu.make_async_copy(rows_v.at[p], out_hbm.at[pl.ds(base, BANK)],
                              ssems[p]).wait()

    issue_bank_gathers(0, 0)

    drain_bank_gathers(0)
    issue_bank_scatter(0, 0)
    issue_bank_gathers(1, 1)

    @pl.loop(1, T - 2, step=2)
    def _(t):
        for d in range(2):
            p = (1 + d) % 2
            q = 1 - p
            drain_bank_gathers(p)
            issue_bank_scatter(t + d, p)
            wait_bank_scatter(q)
            issue_bank_gathers(t + d + 1, q)

    drain_bank_gathers((T - 1) % 2)
    issue_bank_scatter(T - 1, (T - 1) % 2)

    wait_bank_scatter((T - 2) % 2)
    wait_bank_scatter((T - 1) % 2)


@jax.jit
def kernel(x, table):
    x_flat = x.reshape(NW, N_CHUNKS, CHUNK).astype(jnp.int32)
    mesh = plsc.VectorSubcoreMesh(core_axis_name="c", subcore_axis_name="s")
    out = pl.kernel(
        _gather_kernel,
        out_type=jax.ShapeDtypeStruct((B_TOTAL, EMBED_DIM), jnp.float32),
        mesh=mesh,
        scratch_types=[
            pltpu.VMEM((N_CHUNKS, CHUNK), jnp.int32),
            pltpu.VMEM((2, BANK, EMBED_DIM), jnp.float32),
        ] + [pltpu.SemaphoreType.DMA] * 4,
        compiler_params=pltpu.CompilerParams(use_tc_tiling_on_sc=False),
    )(x_flat, table)
    return out.reshape(BATCH, HIST_LEN, EMBED_DIM)
